# Initial kernel scaffold; baseline (speedup 1.0000x reference)
import jax
import jax.numpy as jnp
from jax.experimental import pallas as pl


def kernel(x, table, W, b):
    raise NotImplementedError("write your pallas kernel here")


# speedup vs baseline: 1.6726x; 1.6726x over previous
import functools

import jax
import jax.numpy as jnp
from jax import lax
from jax.experimental import pallas as pl
from jax.experimental.pallas import tpu as pltpu
from jax.experimental.pallas import tpu_sc as plsc

BATCH = 16384
HIST = 50
EMBED_DIM = 16

NC = 2
NS = 16
NW = NC * NS
B_PER_W = BATCH // NW
CB = 64
NCHUNK = B_PER_W // CB
IDX_PER_CHUNK = CB * HIST
GATHER_W = 128
NGATHER = IDX_PER_CHUNK // GATHER_W
IDX_ROWS_PER_W = (B_PER_W * HIST) // GATHER_W


def _shuf(v, perm2d):
    dn = lax.GatherDimensionNumbers(
        offset_dims=(), collapsed_slice_dims=(0,), start_index_map=(0,))
    return lax.gather(v, perm2d, dn, slice_sizes=(1,),
                      mode=lax.GatherScatterMode.PROMISE_IN_BOUNDS)


def _sc_body(x_hbm, table_hbm, w_hbm, b_hbm, out_hbm,
             idx_v, rows_v, out_v, wv_v, bv_v, sem):
    wid = lax.axis_index("s") * NC + lax.axis_index("c")

    pltpu.sync_copy(w_hbm, wv_v)
    pltpu.sync_copy(b_hbm, bv_v)
    wv = wv_v[...]
    bv = bv_v[...]

    def chunk_body(c, carry):
        base = wid * B_PER_W * HIST + c * IDX_PER_CHUNK
        pltpu.sync_copy(x_hbm.at[pl.ds(base, IDX_PER_CHUNK)], idx_v)

        descs = []
        for j in range(NGATHER):
            descs.append(
                pltpu.async_copy(
                    table_hbm.at[idx_v.at[pl.ds(j * GATHER_W, GATHER_W)]],
                    rows_v.at[pl.ds(j * GATHER_W, GATHER_W)],
                    sem,
                )
            )
        for d in descs:
            d.wait()

        lanes = lax.broadcasted_iota(jnp.int32, (16,), 0)
        perms = [((lanes + d) % 16)[:, None] for d in (8, 4, 2, 1)]
        for g in range(CB // 16):
            def batch_body(k, zacc, g=g):
                base = (g * 16 + k) * HIST
                acc0 = rows_v[base + 0, :]
                acc1 = rows_v[base + 1, :]
                acc2 = rows_v[base + 2, :]
                acc3 = rows_v[base + 3, :]
                for r in range(4, HIST, 4):
                    acc0 = acc0 + rows_v[base + r + 0, :]
                    acc1 = acc1 + rows_v[base + r + 1, :]
                    if r + 2 < HIST:
                        acc2 = acc2 + rows_v[base + r + 2, :]
                        acc3 = acc3 + rows_v[base + r + 3, :]
                acc = (acc0 + acc1) + (acc2 + acc3)
                z = acc * wv
                for p in perms:
                    z = z + _shuf(z, p)
                return jnp.where(lanes == k, z, zacc)

            zacc = lax.fori_loop(0, 16, batch_body, jnp.zeros((16,), jnp.float32),
                                 unroll=False)
            zv = zacc + bv
            s = 1.0 / (1.0 + jnp.exp(-zv))
            r4 = (s * 1e4 + 0.5).astype(jnp.int32).astype(jnp.float32) * 1e-4
            out_v[pl.ds(c * CB + g * 16, 16)] = r4
        return carry

    lax.fori_loop(0, NCHUNK, chunk_body, 0, unroll=False)

    pltpu.sync_copy(out_v, out_hbm.at[pl.ds(wid * B_PER_W, B_PER_W)])


@jax.jit
def _embed_bag_sc(x_view, table, wv, bv):
    mesh = plsc.VectorSubcoreMesh(core_axis_name="c", subcore_axis_name="s")
    f = pl.kernel(
        _sc_body,
        out_type=jax.ShapeDtypeStruct((BATCH,), jnp.float32),
        mesh=mesh,
        compiler_params=pltpu.CompilerParams(use_tc_tiling_on_sc=False),
        scratch_types=[
            pltpu.VMEM((IDX_PER_CHUNK,), jnp.int32),
            pltpu.VMEM((IDX_PER_CHUNK, EMBED_DIM), jnp.float32),
            pltpu.VMEM((B_PER_W,), jnp.float32),
            pltpu.VMEM((EMBED_DIM,), jnp.float32),
            pltpu.VMEM((EMBED_DIM,), jnp.float32),
            pltpu.SemaphoreType.DMA,
        ],
    )
    return f(x_view, table, wv, bv)


def kernel(x, table, W, b):
    x_view = x.astype(jnp.int32).reshape(BATCH * HIST)
    wv = (W.reshape(EMBED_DIM) / HIST).astype(jnp.float32)
    bv = jnp.broadcast_to(b.astype(jnp.float32), (EMBED_DIM,))
    out = _embed_bag_sc(x_view, table, wv, bv)
    return out.reshape(BATCH, 1)


# speedup vs baseline: 7.1431x; 4.2706x over previous
import functools

import jax
import jax.numpy as jnp
from jax import lax
from jax.experimental import pallas as pl
from jax.experimental.pallas import tpu as pltpu
from jax.experimental.pallas import tpu_sc as plsc

BATCH = 16384
HIST = 50
EMBED_DIM = 16
VOCAB = 1000000

NC = 2
NS = 16
NW = NC * NS
B_PER_W = BATCH // NW
CB = 128
NCHUNK = B_PER_W // CB

TC_BK = 8192


def _tc_body(w_ref, tt_ref, t_ref):
    t_ref[...] = jnp.sum(tt_ref[...] * w_ref[...], axis=0)


@jax.jit
def _fold_table(tt, wcol):
    grid = (VOCAB + TC_BK - 1) // TC_BK
    return pl.pallas_call(
        _tc_body,
        grid=(grid,),
        in_specs=[
            pl.BlockSpec((EMBED_DIM, 1), lambda i: (0, 0)),
            pl.BlockSpec((EMBED_DIM, TC_BK), lambda i: (0, i)),
        ],
        out_specs=pl.BlockSpec((TC_BK,), lambda i: (i,)),
        out_shape=jax.ShapeDtypeStruct((VOCAB,), jnp.float32),
    )(wcol, tt)


def _sc_body(xt_hbm, t_hbm, bv_hbm, out_hbm,
             idx_v, vals_v, out_v, bv_v, t_sh, sem):
    cid = lax.axis_index("c")
    sid = lax.axis_index("s")
    wid = sid * NC + cid

    @pl.when(sid == 0)
    def _load_t():
        pltpu.sync_copy(t_hbm, t_sh)

    plsc.subcore_barrier()

    pltpu.sync_copy(bv_hbm, bv_v)
    bvec = bv_v[...]

    def chunk_body(c, carry):
        col0 = wid * B_PER_W + c * CB
        pltpu.sync_copy(xt_hbm.at[:, pl.ds(col0, CB)], idx_v)

        descs = []
        for l in range(HIST):
            descs.append(
                pltpu.async_copy(t_sh.at[idx_v.at[l]], vals_v.at[l], sem)
            )
        for d in descs:
            d.wait()

        for j in range(CB // 16):
            acc0 = vals_v[0, pl.ds(j * 16, 16)]
            acc1 = vals_v[1, pl.ds(j * 16, 16)]
            for l in range(2, HIST, 2):
                acc0 = acc0 + vals_v[l, pl.ds(j * 16, 16)]
                acc1 = acc1 + vals_v[l + 1, pl.ds(j * 16, 16)]
            zv = (acc0 + acc1) + bvec
            s = 1.0 / (1.0 + jnp.exp(-zv))
            r4 = (s * 1e4 + 0.5).astype(jnp.int32).astype(jnp.float32) * 1e-4
            out_v[pl.ds(c * CB + j * 16, 16)] = r4
        return carry

    lax.fori_loop(0, NCHUNK, chunk_body, 0, unroll=False)

    pltpu.sync_copy(out_v, out_hbm.at[pl.ds(wid * B_PER_W, B_PER_W)])


@jax.jit
def _embed_bag_sc(xt, t, bv):
    mesh = plsc.VectorSubcoreMesh(core_axis_name="c", subcore_axis_name="s")
    f = pl.kernel(
        _sc_body,
        out_type=jax.ShapeDtypeStruct((BATCH,), jnp.float32),
        mesh=mesh,
        scratch_types=[
            pltpu.VMEM((HIST, CB), jnp.int32),
            pltpu.VMEM((HIST, CB), jnp.float32),
            pltpu.VMEM((B_PER_W,), jnp.float32),
            pltpu.VMEM((EMBED_DIM,), jnp.float32),
            pltpu.VMEM_SHARED((VOCAB,), jnp.float32),
            pltpu.SemaphoreType.DMA,
        ],
    )
    return f(xt, t, bv)


def kernel(x, table, W, b):
    xt = x.astype(jnp.int32).T
    tt = table.T
    wcol = (W.reshape(EMBED_DIM, 1) / HIST).astype(jnp.float32)
    t = _fold_table(tt, wcol)
    bv = jnp.broadcast_to(b.astype(jnp.float32), (EMBED_DIM,))
    out = _embed_bag_sc(xt, t, bv)
    return out.reshape(BATCH, 1)


# speedup vs baseline: 7.4424x; 1.0419x over previous
import functools

import jax
import jax.numpy as jnp
from jax import lax
from jax.experimental import pallas as pl
from jax.experimental.pallas import tpu as pltpu
from jax.experimental.pallas import tpu_sc as plsc

BATCH = 16384
HIST = 50
EMBED_DIM = 16
VOCAB = 1000000

NC = 2
NS = 16
NW = NC * NS
B_PER_W = BATCH // NW
CB = 128
NCHUNK = B_PER_W // CB

TC_BK = 32768


def _tc_body(w_ref, tt_ref, t_ref):
    t_ref[...] = jnp.dot(w_ref[...], tt_ref[...],
                         preferred_element_type=jnp.float32)


@jax.jit
def _fold_table(tt, wrow):
    grid = (VOCAB + TC_BK - 1) // TC_BK
    out = pl.pallas_call(
        _tc_body,
        grid=(grid,),
        in_specs=[
            pl.BlockSpec((1, EMBED_DIM), lambda i: (0, 0)),
            pl.BlockSpec((EMBED_DIM, TC_BK), lambda i: (0, i)),
        ],
        out_specs=pl.BlockSpec((1, TC_BK), lambda i: (0, i)),
        out_shape=jax.ShapeDtypeStruct((1, VOCAB), jnp.float32),
    )(wrow, tt)
    return out.reshape(VOCAB)


def _sc_body(xt_hbm, t_hbm, bv_hbm, out_hbm,
             idx_v, vals_v, out_v, bv_v, t_sh, sem):
    cid = lax.axis_index("c")
    sid = lax.axis_index("s")
    wid = sid * NC + cid

    @pl.when(sid == 0)
    def _load_t():
        pltpu.sync_copy(t_hbm, t_sh)

    plsc.subcore_barrier()

    pltpu.sync_copy(bv_hbm, bv_v)
    bvec = bv_v[...]

    def chunk_body(c, carry):
        col0 = wid * B_PER_W + c * CB
        pltpu.sync_copy(xt_hbm.at[:, pl.ds(col0, CB)], idx_v)

        descs = []
        for l in range(HIST):
            descs.append(
                pltpu.async_copy(t_sh.at[idx_v.at[l]], vals_v.at[l], sem)
            )
        for d in descs:
            d.wait()

        for j in range(CB // 16):
            acc0 = vals_v[0, pl.ds(j * 16, 16)]
            acc1 = vals_v[1, pl.ds(j * 16, 16)]
            for l in range(2, HIST, 2):
                acc0 = acc0 + vals_v[l, pl.ds(j * 16, 16)]
                acc1 = acc1 + vals_v[l + 1, pl.ds(j * 16, 16)]
            zv = (acc0 + acc1) + bvec
            s = 1.0 / (1.0 + jnp.exp(-zv))
            r4 = (s * 1e4 + 0.5).astype(jnp.int32).astype(jnp.float32) * 1e-4
            out_v[pl.ds(c * CB + j * 16, 16)] = r4
        return carry

    lax.fori_loop(0, NCHUNK, chunk_body, 0, unroll=False)

    pltpu.sync_copy(out_v, out_hbm.at[pl.ds(wid * B_PER_W, B_PER_W)])


@jax.jit
def _embed_bag_sc(xt, t, bv):
    mesh = plsc.VectorSubcoreMesh(core_axis_name="c", subcore_axis_name="s")
    f = pl.kernel(
        _sc_body,
        out_type=jax.ShapeDtypeStruct((BATCH,), jnp.float32),
        mesh=mesh,
        scratch_types=[
            pltpu.VMEM((HIST, CB), jnp.int32),
            pltpu.VMEM((HIST, CB), jnp.float32),
            pltpu.VMEM((B_PER_W,), jnp.float32),
            pltpu.VMEM((EMBED_DIM,), jnp.float32),
            pltpu.VMEM_SHARED((VOCAB,), jnp.float32),
            pltpu.SemaphoreType.DMA,
        ],
    )
    return f(xt, t, bv)


def kernel(x, table, W, b):
    xt = x.astype(jnp.int32).T
    tt = table.T
    wrow = (W.reshape(1, EMBED_DIM) / HIST).astype(jnp.float32)
    t = _fold_table(tt, wrow)
    bv = jnp.broadcast_to(b.astype(jnp.float32), (EMBED_DIM,))
    out = _embed_bag_sc(xt, t, bv)
    return out.reshape(BATCH, 1)


# speedup vs baseline: 11.8440x; 1.5914x over previous
import functools

import jax
import jax.numpy as jnp
from jax import lax
from jax.experimental import pallas as pl
from jax.experimental.pallas import tpu as pltpu
from jax.experimental.pallas import tpu_sc as plsc

BATCH = 16384
HIST = 50
EMBED_DIM = 16
VOCAB = 1000000

NC = 2
NS = 16
NW = NC * NS
B_PER_W = BATCH // NW
CB = 128
NCHUNK = B_PER_W // CB

TC_BK = 32768


def _tc_body(w_ref, tt_ref, t_ref):
    t_ref[...] = jnp.dot(w_ref[...], tt_ref[...],
                         preferred_element_type=jnp.float32)[0]


@jax.jit
def _fold_table(tt, wrow):
    grid = (VOCAB + TC_BK - 1) // TC_BK
    return pl.pallas_call(
        _tc_body,
        grid=(grid,),
        in_specs=[
            pl.BlockSpec((1, EMBED_DIM), lambda i: (0, 0)),
            pl.BlockSpec((EMBED_DIM, TC_BK), lambda i: (0, i)),
        ],
        out_specs=pl.BlockSpec((TC_BK,), lambda i: (i,)),
        out_shape=jax.ShapeDtypeStruct((VOCAB,), jnp.float32),
    )(wrow, tt)


def _sc_body(xt_hbm, t_hbm, bv_hbm, out_hbm,
             idx_v, vals_v, out_v, bv_v, t_sh, sem):
    cid = lax.axis_index("c")
    sid = lax.axis_index("s")
    wid = sid * NC + cid

    @pl.when(sid == 0)
    def _load_t():
        pltpu.sync_copy(t_hbm, t_sh)

    plsc.subcore_barrier()

    pltpu.sync_copy(bv_hbm, bv_v)
    bvec = bv_v[...]

    def chunk_body(c, carry):
        col0 = wid * B_PER_W + c * CB
        pltpu.sync_copy(xt_hbm.at[:, pl.ds(col0, CB)], idx_v)

        descs = []
        for l in range(HIST):
            descs.append(
                pltpu.async_copy(t_sh.at[idx_v.at[l]], vals_v.at[l], sem)
            )
        for d in descs:
            d.wait()

        for j in range(CB // 16):
            acc0 = vals_v[0, pl.ds(j * 16, 16)]
            acc1 = vals_v[1, pl.ds(j * 16, 16)]
            for l in range(2, HIST, 2):
                acc0 = acc0 + vals_v[l, pl.ds(j * 16, 16)]
                acc1 = acc1 + vals_v[l + 1, pl.ds(j * 16, 16)]
            zv = (acc0 + acc1) + bvec
            s = 1.0 / (1.0 + jnp.exp(-zv))
            r4 = (s * 1e4 + 0.5).astype(jnp.int32).astype(jnp.float32) * 1e-4
            out_v[pl.ds(c * CB + j * 16, 16)] = r4
        return carry

    lax.fori_loop(0, NCHUNK, chunk_body, 0, unroll=False)

    pltpu.sync_copy(out_v, out_hbm.at[pl.ds(wid * B_PER_W, B_PER_W)])


@jax.jit
def _embed_bag_sc(xt, t, bv):
    mesh = plsc.VectorSubcoreMesh(core_axis_name="c", subcore_axis_name="s")
    f = pl.kernel(
        _sc_body,
        out_type=jax.ShapeDtypeStruct((BATCH,), jnp.float32),
        mesh=mesh,
        scratch_types=[
            pltpu.VMEM((HIST, CB), jnp.int32),
            pltpu.VMEM((HIST, CB), jnp.float32),
            pltpu.VMEM((B_PER_W,), jnp.float32),
            pltpu.VMEM((EMBED_DIM,), jnp.float32),
            pltpu.VMEM_SHARED((VOCAB,), jnp.float32),
            pltpu.SemaphoreType.DMA,
        ],
    )
    return f(xt, t, bv)


def kernel(x, table, W, b):
    xt = x.astype(jnp.int32).T
    tt = table.T
    wrow = (W.reshape(1, EMBED_DIM) / HIST).astype(jnp.float32)
    t = _fold_table(tt, wrow)
    bv = jnp.broadcast_to(b.astype(jnp.float32), (EMBED_DIM,))
    out = _embed_bag_sc(xt, t, bv)
    return out.reshape(BATCH, 1)


# speedup vs baseline: 13.4684x; 1.1371x over previous
import functools

import jax
import jax.numpy as jnp
from jax import lax
from jax.experimental import pallas as pl
from jax.experimental.pallas import tpu as pltpu
from jax.experimental.pallas import tpu_sc as plsc

BATCH = 16384
HIST = 50
EMBED_DIM = 16
VOCAB = 1000000

NC = 2
NS = 16
NW = NC * NS
B_PER_W = BATCH // NW
CB = 128
NCHUNK = B_PER_W // CB

TC_BK = 65536


def _tc_body(w_ref, tt_ref, t_ref):
    t_ref[...] = jnp.dot(w_ref[...], tt_ref[...],
                         preferred_element_type=jnp.float32)[0]


@jax.jit
def _fold_table(tt, wrow):
    grid = (VOCAB + TC_BK - 1) // TC_BK
    return pl.pallas_call(
        _tc_body,
        grid=(grid,),
        in_specs=[
            pl.BlockSpec((1, EMBED_DIM), lambda i: (0, 0)),
            pl.BlockSpec((EMBED_DIM, TC_BK), lambda i: (0, i)),
        ],
        out_specs=pl.BlockSpec((TC_BK,), lambda i: (i,)),
        out_shape=jax.ShapeDtypeStruct((VOCAB,), jnp.float32),
    )(wrow, tt)


def _sc_body(xt_hbm, t_hbm, bv_hbm, out_hbm,
             idx_v, vals_v, out_v, bv_v, t_sh, sem):
    cid = lax.axis_index("c")
    sid = lax.axis_index("s")
    wid = sid * NC + cid

    @pl.when(sid == 0)
    def _load_t():
        pltpu.sync_copy(t_hbm, t_sh)

    plsc.subcore_barrier()

    pltpu.sync_copy(bv_hbm, bv_v)
    bvec = bv_v[...]

    def chunk_body(c, carry):
        col0 = wid * B_PER_W + c * CB
        pltpu.sync_copy(xt_hbm.at[:, pl.ds(col0, CB)], idx_v)

        descs = []
        for l in range(HIST):
            descs.append(
                pltpu.async_copy(t_sh.at[idx_v.at[l]], vals_v.at[l], sem)
            )
        for d in descs:
            d.wait()

        for j in range(CB // 16):
            acc0 = vals_v[0, pl.ds(j * 16, 16)]
            acc1 = vals_v[1, pl.ds(j * 16, 16)]
            for l in range(2, HIST, 2):
                acc0 = acc0 + vals_v[l, pl.ds(j * 16, 16)]
                acc1 = acc1 + vals_v[l + 1, pl.ds(j * 16, 16)]
            zv = (acc0 + acc1) + bvec
            s = 1.0 / (1.0 + jnp.exp(-zv))
            r4 = (s * 1e4 + 0.5).astype(jnp.int32).astype(jnp.float32) * 1e-4
            out_v[pl.ds(c * CB + j * 16, 16)] = r4
        return carry

    lax.fori_loop(0, NCHUNK, chunk_body, 0, unroll=False)

    pltpu.sync_copy(out_v, out_hbm.at[pl.ds(wid * B_PER_W, B_PER_W)])


@jax.jit
def _embed_bag_sc(xt, t, bv):
    mesh = plsc.VectorSubcoreMesh(core_axis_name="c", subcore_axis_name="s")
    f = pl.kernel(
        _sc_body,
        out_type=jax.ShapeDtypeStruct((BATCH,), jnp.float32),
        mesh=mesh,
        scratch_types=[
            pltpu.VMEM((HIST, CB), jnp.int32),
            pltpu.VMEM((HIST, CB), jnp.float32),
            pltpu.VMEM((B_PER_W,), jnp.float32),
            pltpu.VMEM((EMBED_DIM,), jnp.float32),
            pltpu.VMEM_SHARED((VOCAB,), jnp.float32),
            pltpu.SemaphoreType.DMA,
        ],
    )
    return f(xt, t, bv)


def kernel(x, table, W, b):
    xt = x.astype(jnp.int32).T
    tt = table.T
    wrow = (W.reshape(1, EMBED_DIM) / HIST).astype(jnp.float32)
    t = _fold_table(tt, wrow)
    bv = jnp.broadcast_to(b.astype(jnp.float32), (EMBED_DIM,))
    out = _embed_bag_sc(xt, t, bv)
    return out.reshape(BATCH, 1)


# speedup vs baseline: 13.8662x; 1.0295x over previous
import functools

import jax
import jax.numpy as jnp
from jax import lax
from jax.experimental import pallas as pl
from jax.experimental.pallas import tpu as pltpu
from jax.experimental.pallas import tpu_sc as plsc

BATCH = 16384
HIST = 50
EMBED_DIM = 16
VOCAB = 1000000

NC = 2
NS = 16
NW = NC * NS
B_PER_W = BATCH // NW
CB = 128
NCHUNK = B_PER_W // CB

TC_BK = 131072


def _tc_body(w_ref, tt_ref, t_ref):
    t_ref[...] = jnp.dot(w_ref[...], tt_ref[...],
                         preferred_element_type=jnp.float32)[0]


@jax.jit
def _fold_table(tt, wrow):
    grid = (VOCAB + TC_BK - 1) // TC_BK
    return pl.pallas_call(
        _tc_body,
        grid=(grid,),
        in_specs=[
            pl.BlockSpec((1, EMBED_DIM), lambda i: (0, 0)),
            pl.BlockSpec((EMBED_DIM, TC_BK), lambda i: (0, i)),
        ],
        out_specs=pl.BlockSpec((TC_BK,), lambda i: (i,)),
        out_shape=jax.ShapeDtypeStruct((VOCAB,), jnp.float32),
    )(wrow, tt)


def _sc_body(xt_hbm, t_hbm, bv_hbm, out_hbm,
             idx_v, vals_v, out_v, bv_v, t_sh, sem):
    cid = lax.axis_index("c")
    sid = lax.axis_index("s")
    wid = sid * NC + cid

    @pl.when(sid == 0)
    def _load_t():
        pltpu.sync_copy(t_hbm, t_sh)

    plsc.subcore_barrier()

    pltpu.sync_copy(bv_hbm, bv_v)
    bvec = bv_v[...]

    def chunk_body(c, carry):
        col0 = wid * B_PER_W + c * CB
        pltpu.sync_copy(xt_hbm.at[:, pl.ds(col0, CB)], idx_v)

        descs = []
        for l in range(HIST):
            descs.append(
                pltpu.async_copy(t_sh.at[idx_v.at[l]], vals_v.at[l], sem)
            )
        for d in descs:
            d.wait()

        for j in range(CB // 16):
            acc0 = vals_v[0, pl.ds(j * 16, 16)]
            acc1 = vals_v[1, pl.ds(j * 16, 16)]
            for l in range(2, HIST, 2):
                acc0 = acc0 + vals_v[l, pl.ds(j * 16, 16)]
                acc1 = acc1 + vals_v[l + 1, pl.ds(j * 16, 16)]
            zv = (acc0 + acc1) + bvec
            s = 1.0 / (1.0 + jnp.exp(-zv))
            r4 = (s * 1e4 + 0.5).astype(jnp.int32).astype(jnp.float32) * 1e-4
            out_v[pl.ds(c * CB + j * 16, 16)] = r4
        return carry

    lax.fori_loop(0, NCHUNK, chunk_body, 0, unroll=False)

    pltpu.sync_copy(out_v, out_hbm.at[pl.ds(wid * B_PER_W, B_PER_W)])


@jax.jit
def _embed_bag_sc(xt, t, bv):
    mesh = plsc.VectorSubcoreMesh(core_axis_name="c", subcore_axis_name="s")
    f = pl.kernel(
        _sc_body,
        out_type=jax.ShapeDtypeStruct((BATCH,), jnp.float32),
        mesh=mesh,
        scratch_types=[
            pltpu.VMEM((HIST, CB), jnp.int32),
            pltpu.VMEM((HIST, CB), jnp.float32),
            pltpu.VMEM((B_PER_W,), jnp.float32),
            pltpu.VMEM((EMBED_DIM,), jnp.float32),
            pltpu.VMEM_SHARED((VOCAB,), jnp.float32),
            pltpu.SemaphoreType.DMA,
        ],
    )
    return f(xt, t, bv)


def kernel(x, table, W, b):
    xt = x.astype(jnp.int32).T
    tt = table.T
    wrow = (W.reshape(1, EMBED_DIM) / HIST).astype(jnp.float32)
    t = _fold_table(tt, wrow)
    bv = jnp.broadcast_to(b.astype(jnp.float32), (EMBED_DIM,))
    out = _embed_bag_sc(xt, t, bv)
    return out.reshape(BATCH, 1)


# speedup vs baseline: 14.4444x; 1.0417x over previous
import functools

import jax
import jax.numpy as jnp
from jax import lax
from jax.experimental import pallas as pl
from jax.experimental.pallas import tpu as pltpu
from jax.experimental.pallas import tpu_sc as plsc

BATCH = 16384
HIST = 50
EMBED_DIM = 16
VOCAB = 1000000

NC = 2
NS = 16
NW = NC * NS
B_PER_W = BATCH // NW
CB = 128
NCHUNK = B_PER_W // CB

TC_BK = 131072


def _tc_body(w_ref, tt_ref, t_ref):
    t_ref[...] = jnp.dot(w_ref[...], tt_ref[...],
                         preferred_element_type=jnp.float32)[0]


@jax.jit
def _fold_table(tt, wrow):
    grid = (VOCAB + TC_BK - 1) // TC_BK
    return pl.pallas_call(
        _tc_body,
        grid=(grid,),
        in_specs=[
            pl.BlockSpec((1, EMBED_DIM), lambda i: (0, 0)),
            pl.BlockSpec((EMBED_DIM, TC_BK), lambda i: (0, i)),
        ],
        out_specs=pl.BlockSpec((TC_BK,), lambda i: (i,)),
        out_shape=jax.ShapeDtypeStruct((VOCAB,), jnp.float32),
    )(wrow, tt)


def _sc_body(xt_hbm, t_hbm, bv_hbm, out_hbm,
             idx_v, vals_v, out_v, bv_v, t_sh, sem, isem):
    cid = lax.axis_index("c")
    sid = lax.axis_index("s")
    wid = sid * NC + cid

    idescs = []
    for c in range(NCHUNK):
        col0 = wid * B_PER_W + c * CB
        idescs.append(
            pltpu.async_copy(xt_hbm.at[:, pl.ds(col0, CB)], idx_v.at[c], isem)
        )
    pltpu.sync_copy(bv_hbm, bv_v)
    bvec = bv_v[...]

    @pl.when(sid == 0)
    def _load_t():
        pltpu.sync_copy(t_hbm, t_sh)

    for d in idescs:
        d.wait()
    plsc.subcore_barrier()

    def chunk_body(c, carry):
        descs = []
        for l in range(HIST):
            descs.append(
                pltpu.async_copy(t_sh.at[idx_v.at[c, l]], vals_v.at[l], sem)
            )
        for d in descs:
            d.wait()

        for j in range(CB // 16):
            acc0 = vals_v[0, pl.ds(j * 16, 16)]
            acc1 = vals_v[1, pl.ds(j * 16, 16)]
            for l in range(2, HIST, 2):
                acc0 = acc0 + vals_v[l, pl.ds(j * 16, 16)]
                acc1 = acc1 + vals_v[l + 1, pl.ds(j * 16, 16)]
            zv = (acc0 + acc1) + bvec
            s = 1.0 / (1.0 + jnp.exp(-zv))
            r4 = (s * 1e4 + 0.5).astype(jnp.int32).astype(jnp.float32) * 1e-4
            out_v[pl.ds(c * CB + j * 16, 16)] = r4
        return carry

    lax.fori_loop(0, NCHUNK, chunk_body, 0, unroll=False)

    pltpu.sync_copy(out_v, out_hbm.at[pl.ds(wid * B_PER_W, B_PER_W)])


@jax.jit
def _embed_bag_sc(xt, t, bv):
    mesh = plsc.VectorSubcoreMesh(core_axis_name="c", subcore_axis_name="s")
    f = pl.kernel(
        _sc_body,
        out_type=jax.ShapeDtypeStruct((BATCH,), jnp.float32),
        mesh=mesh,
        scratch_types=[
            pltpu.VMEM((NCHUNK, HIST, CB), jnp.int32),
            pltpu.VMEM((HIST, CB), jnp.float32),
            pltpu.VMEM((B_PER_W,), jnp.float32),
            pltpu.VMEM((EMBED_DIM,), jnp.float32),
            pltpu.VMEM_SHARED((VOCAB,), jnp.float32),
            pltpu.SemaphoreType.DMA,
            pltpu.SemaphoreType.DMA,
        ],
    )
    return f(xt, t, bv)


def kernel(x, table, W, b):
    xt = x.astype(jnp.int32).T
    tt = table.T
    wrow = (W.reshape(1, EMBED_DIM) / HIST).astype(jnp.float32)
    t = _fold_table(tt, wrow)
    bv = jnp.broadcast_to(b.astype(jnp.float32), (EMBED_DIM,))
    out = _embed_bag_sc(xt, t, bv)
    return out.reshape(BATCH, 1)
